# baseline (speedup 1.0000x reference)
import functools

import jax
import jax.numpy as jnp
from jax import lax
from jax.experimental import pallas as pl
from jax.experimental.pallas import tpu as pltpu
from jax.experimental.pallas import tpu_sc as plsc

_N = 10000
_E = 160000
_F = 128
_HID = 32
_HEADS = 8
_NCLS = 2
_PHID = 64

_NSUB = 16
_NCORE = 2
_BE1 = 48
_BEC = 64
_NPAD = 10112
_ROWS_PER_TILE = _NPAD // _NSUB
_EG = _E + _N
_E1PAD = 172032
_ECPAD = 163840


def _mm_body(x_ref, w_ref, o_ref):
    o_ref[...] = jnp.dot(x_ref[...], w_ref[...],
                         preferred_element_type=jnp.float32)


def _mm(x, w, bm=1024):
    m, k = x.shape
    n = w.shape[1]
    return pl.pallas_call(
        _mm_body,
        grid=(pl.cdiv(m, bm),),
        in_specs=[pl.BlockSpec((bm, k), lambda i: (i, 0)),
                  pl.BlockSpec((k, n), lambda i: (0, 0))],
        out_specs=pl.BlockSpec((bm, n), lambda i: (i, 0)),
        out_shape=jax.ShapeDtypeStruct((m, n), jnp.float32),
    )(x, w)


def _gat_chunks(src_hbm, dst_hbm, xl_hbm, xr_hbm, att_ref, accS, bufs,
                tile_base, nchunks, hpc, chans, be):
    w = hpc * chans
    nv = chans // 16
    srcv, dstv, xlb, xrb, mzb, sgl, sgr, ss = bufs
    iota16 = lax.iota(jnp.int32, 16)
    perms = [(iota16 ^ s).reshape(16, 1) for s in (8, 4, 2, 1)]
    gdn = lax.GatherDimensionNumbers(
        offset_dims=(), collapsed_slice_dims=(0,), start_index_map=(0,))

    def _lane_shuffle(v, p):
        return lax.gather(v, p, gdn, slice_sizes=(1,),
                          mode=lax.GatherScatterMode.PROMISE_IN_BOUNDS)

    att_vecs = [att_ref[pl.ds(k * 16, 16)] for k in range(w // 16)]

    def make_edge_body(b):
        def edge_body(e):
            zvec = jnp.zeros((16,), jnp.float32)
            for h in range(hpc):
                xs = []
                tot = None
                for k in range(nv):
                    off = h * chans + k * 16
                    xv = xlb[b][e, pl.ds(off, 16)]
                    rv = xrb[b][e, pl.ds(off, 16)]
                    t = xv + rv
                    t = jnp.where(t >= 0, t, t * jnp.float32(0.2))
                    t = t * att_vecs[off // 16]
                    xs.append(xv)
                    tot = t if tot is None else tot + t
                for p in perms:
                    tot = tot + _lane_shuffle(tot, p)
                sv = jnp.exp(tot)
                for k in range(nv):
                    off = h * chans + k * 16
                    mzb[b][e, pl.ds(off, 16)] = sv * xs[k]
                zvec = jnp.where(iota16 == h, sv, zvec)
            mzb[b][e, pl.ds(w, 16)] = zvec
        return edge_body

    def prefetch(j, b):
        base = tile_base + j * be
        pltpu.sync_copy(src_hbm.at[pl.ds(base, be)], srcv[b])
        pltpu.sync_copy(dst_hbm.at[pl.ds(base, be)], dstv[b])
        pltpu.async_copy(xl_hbm.at[srcv[b]], xlb[b], sgl[b])
        pltpu.async_copy(xr_hbm.at[dstv[b]], xrb[b], sgr[b])

    prefetch(0, 0)

    def pair_body(jj, carry):
        for b in range(2):
            j = jj * 2 + b
            @pl.when(j >= 1)
            def _():
                pltpu.make_async_copy(mzb[1 - b], accS.at[dstv[1 - b]],
                                      ss[1 - b]).wait()

            @pl.when(j + 1 < nchunks)
            def _():
                prefetch(j + 1, 1 - b)

            pltpu.make_async_copy(xl_hbm.at[srcv[b]], xlb[b], sgl[b]).wait()
            pltpu.make_async_copy(xr_hbm.at[dstv[b]], xrb[b], sgr[b]).wait()
            plsc.parallel_loop(0, be, 1, unroll=8)(make_edge_body(b))
            pltpu.async_copy(mzb[b], accS.at[dstv[b]], ss[b], add=True)
        return carry

    lax.fori_loop(0, nchunks // 2, pair_body, 0)
    pltpu.make_async_copy(mzb[1], accS.at[dstv[1]], ss[1]).wait()


def _gat_core(src_hbm, dst_hbm, xl_hbm, xr_hbm, att_hbm, zero_hbm, out_hbm,
              bufs, attv, accS, sid, tile_base, nchunks, hpc, chans, be):
    r0 = sid * _ROWS_PER_TILE
    pltpu.sync_copy(zero_hbm.at[pl.ds(r0, _ROWS_PER_TILE)],
                    accS.at[pl.ds(r0, _ROWS_PER_TILE)])
    pltpu.sync_copy(att_hbm, attv)
    plsc.subcore_barrier()
    _gat_chunks(src_hbm, dst_hbm, xl_hbm, xr_hbm, attv, accS, bufs,
                tile_base, nchunks, hpc, chans, be)
    plsc.subcore_barrier()
    pltpu.sync_copy(accS.at[pl.ds(r0, _ROWS_PER_TILE)],
                    out_hbm.at[pl.ds(r0, _ROWS_PER_TILE)])


def _gat_scratch(w):
    dbl = lambda t: [t, t]
    return (dbl(pltpu.VMEM((_BE1,), jnp.int32)) +
            dbl(pltpu.VMEM((_BE1,), jnp.int32)) +
            dbl(pltpu.VMEM((_BE1, w), jnp.float32)) +
            dbl(pltpu.VMEM((_BE1, w), jnp.float32)) +
            dbl(pltpu.VMEM((_BE1, w + 16), jnp.float32)) +
            [pltpu.VMEM((w,), jnp.float32),
             pltpu.VMEM_SHARED((_NPAD, w + 16), jnp.float32)] +
            [pltpu.SemaphoreType.DMA] * 6)


def _pack_bufs(args):
    (s0, s1, d0, d1, xl0, xl1, xr0, xr1, m0, m1,
     attv, accS, g0, g1, g2, g3, ss0, ss1) = args
    bufs = ([s0, s1], [d0, d1], [xl0, xl1], [xr0, xr1], [m0, m1],
            [g0, g1], [g2, g3], [ss0, ss1])
    return bufs, attv, accS


def _sc_gat1(xl0, xl1, xr0, xr1, att0, att1, srcp, dstp, zeros):
    hpc, chans = _HEADS // _NCORE, _HID
    w = hpc * chans
    nchunks = _E1PAD // _NSUB // _BE1
    mesh = plsc.VectorSubcoreMesh(core_axis_name="c", subcore_axis_name="s")

    @functools.partial(
        pl.kernel,
        out_type=[jax.ShapeDtypeStruct((_NPAD, w + 16), jnp.float32),
                  jax.ShapeDtypeStruct((_NPAD, w + 16), jnp.float32)],
        mesh=mesh,
        compiler_params=pltpu.CompilerParams(use_tc_tiling_on_sc=False,
                                             needs_layout_passes=False),
        scratch_types=_gat_scratch(w),
    )
    def k(src_hbm, dst_hbm, xl0_h, xl1_h, xr0_h, xr1_h, a0_h, a1_h, z_h,
          out0, out1, *scratch):
        bufs, attv, accS = _pack_bufs(scratch)
        cid = lax.axis_index("c")
        sid = lax.axis_index("s")
        tile_base = sid * (nchunks * _BE1)

        @pl.when(cid == 0)
        def _():
            _gat_core(src_hbm, dst_hbm, xl0_h, xr0_h, a0_h, z_h, out0,
                      bufs, attv, accS, sid, tile_base, nchunks, hpc, chans,
                      _BE1)

        @pl.when(cid == 1)
        def _():
            _gat_core(src_hbm, dst_hbm, xl1_h, xr1_h, a1_h, z_h, out1,
                      bufs, attv, accS, sid, tile_base, nchunks, hpc, chans,
                      _BE1)

    return k(srcp, dstp, xl0, xl1, xr0, xr1, att0, att1, zeros)


def _sc_gat2(xl, xr, att, srcp, dstp, zeros):
    hpc, chans = 1, _HID
    w = hpc * chans
    nchunks = _E1PAD // (_NSUB * _NCORE) // _BE1
    mesh = plsc.VectorSubcoreMesh(core_axis_name="c", subcore_axis_name="s")

    @functools.partial(
        pl.kernel,
        out_type=[jax.ShapeDtypeStruct((_NPAD, w + 16), jnp.float32),
                  jax.ShapeDtypeStruct((_NPAD, w + 16), jnp.float32)],
        mesh=mesh,
        compiler_params=pltpu.CompilerParams(use_tc_tiling_on_sc=False,
                                             needs_layout_passes=False),
        scratch_types=_gat_scratch(w),
    )
    def k(src_hbm, dst_hbm, xl_h, xr_h, a_h, z_h, out0, out1, *scratch):
        bufs, attv, accS = _pack_bufs(scratch)
        cid = lax.axis_index("c")
        sid = lax.axis_index("s")
        tile_base = (cid * _NSUB + sid) * (nchunks * _BE1)

        @pl.when(cid == 0)
        def _():
            _gat_core(src_hbm, dst_hbm, xl_h, xr_h, a_h, z_h, out0,
                      bufs, attv, accS, sid, tile_base, nchunks, hpc, chans,
                      _BE1)

        @pl.when(cid == 1)
        def _():
            _gat_core(src_hbm, dst_hbm, xl_h, xr_h, a_h, z_h, out1,
                      bufs, attv, accS, sid, tile_base, nchunks, hpc, chans,
                      _BE1)

    return k(srcp, dstp, xl, xr, att, zeros)


def _sc_crf(t, srcc, dstc, zeros):
    w = 16
    nchunks = _ECPAD // (_NSUB * _NCORE) // _BEC
    mesh = plsc.VectorSubcoreMesh(core_axis_name="c", subcore_axis_name="s")

    @functools.partial(
        pl.kernel,
        out_type=[jax.ShapeDtypeStruct((_NPAD, w), jnp.float32),
                  jax.ShapeDtypeStruct((_NPAD, w), jnp.float32)],
        mesh=mesh,
        compiler_params=pltpu.CompilerParams(use_tc_tiling_on_sc=False,
                                             needs_layout_passes=False),
        scratch_types=[
            pltpu.VMEM((_BEC,), jnp.int32), pltpu.VMEM((_BEC,), jnp.int32),
            pltpu.VMEM((_BEC,), jnp.int32), pltpu.VMEM((_BEC,), jnp.int32),
            pltpu.VMEM((_BEC, w), jnp.float32),
            pltpu.VMEM((_BEC, w), jnp.float32),
            pltpu.VMEM_SHARED((_NPAD, w), jnp.float32),
        ] + [pltpu.SemaphoreType.DMA] * 4,
    )
    def k(src_hbm, dst_hbm, t_h, z_h, out0, out1,
          s0, s1, d0, d1, b0, b1, accS, g0, g1, ss0, ss1):
        srcv, dstv, buf = [s0, s1], [d0, d1], [b0, b1]
        sg, ss = [g0, g1], [ss0, ss1]
        cid = lax.axis_index("c")
        sid = lax.axis_index("s")
        tile_base = (cid * _NSUB + sid) * (nchunks * _BEC)
        r0 = sid * _ROWS_PER_TILE

        def prefetch(j, b):
            base = tile_base + j * _BEC
            pltpu.sync_copy(src_hbm.at[pl.ds(base, _BEC)], srcv[b])
            pltpu.sync_copy(dst_hbm.at[pl.ds(base, _BEC)], dstv[b])
            pltpu.async_copy(t_h.at[srcv[b]], buf[b], sg[b])

        def core(out_hbm):
            pltpu.sync_copy(z_h.at[pl.ds(r0, _ROWS_PER_TILE)],
                            accS.at[pl.ds(r0, _ROWS_PER_TILE)])
            plsc.subcore_barrier()
            prefetch(0, 0)

            def pair_body(jj, carry):
                for b in range(2):
                    j = jj * 2 + b

                    @pl.when(j >= 1)
                    def _():
                        pltpu.make_async_copy(buf[1 - b],
                                              accS.at[dstv[1 - b]],
                                              ss[1 - b]).wait()

                    @pl.when(j + 1 < nchunks)
                    def _():
                        prefetch(j + 1, 1 - b)

                    pltpu.make_async_copy(t_h.at[srcv[b]], buf[b],
                                          sg[b]).wait()
                    pltpu.async_copy(buf[b], accS.at[dstv[b]], ss[b],
                                     add=True)
                return carry

            lax.fori_loop(0, nchunks // 2, pair_body, 0)
            pltpu.make_async_copy(buf[1], accS.at[dstv[1]], ss[1]).wait()
            plsc.subcore_barrier()
            pltpu.sync_copy(accS.at[pl.ds(r0, _ROWS_PER_TILE)],
                            out_hbm.at[pl.ds(r0, _ROWS_PER_TILE)])

        @pl.when(cid == 0)
        def _():
            core(out0)

        @pl.when(cid == 1)
        def _():
            core(out1)

    return k(srcc, dstc, t, zeros)


def _pad_rows(a):
    return jnp.pad(a, ((0, _NPAD - a.shape[0]), (0, 0)))


def _gat_layer1(proj, a1, b1, srcp, dstp, zeros1):
    xl0 = proj[:, 0:128]
    xl1 = proj[:, 128:256]
    xr0 = proj[:, 256:384]
    xr1 = proj[:, 384:512]
    att0 = a1[:4].reshape(-1)
    att1 = a1[4:].reshape(-1)
    o0, o1 = _sc_gat1(xl0, xl1, xr0, xr1, att0, att1, srcp, dstp, zeros1)
    acc = jnp.concatenate([o0[:, :128], o1[:, :128]], axis=1)
    z = jnp.concatenate([o0[:, 128:132], o1[:, 128:132]], axis=1)
    zr = jnp.repeat(z, _HID, axis=1)
    return jax.nn.elu(acc / (zr + 1e-16) + b1)


def _gat_layer2(proj, a2, b2, srcp, dstp, zeros2):
    xl = proj[:, :_HID]
    xr = proj[:, _HID:]
    o0, o1 = _sc_gat2(xl, xr, a2.reshape(-1), srcp, dstp, zeros2)
    s = o0 + o1
    return s[:, :_HID] / (s[:, _HID:_HID + 1] + 1e-16) + b2


def _gnn(xp, p, srcp, dstp, zeros1, zeros2):
    p1 = _mm(xp, jnp.concatenate([p['Wl1'], p['Wr1']], axis=1))
    h1 = _gat_layer1(p1, p['a1'], p['b1'], srcp, dstp, zeros1)
    p2 = _mm(h1, jnp.concatenate([p['Wl2'], p['Wr2']], axis=1))
    return _gat_layer2(p2, p['a2'], p['b2'], srcp, dstp, zeros2)


def kernel(x, edge_type, edge_index, params):
    src = edge_index[0]
    dst = edge_index[1]
    si = jnp.arange(_N, dtype=src.dtype)
    padv = jnp.full((_E1PAD - _EG,), _N, src.dtype)
    srcp = jnp.concatenate([src, si, padv])
    dstp = jnp.concatenate([dst, si, padv])
    padc = jnp.full((_ECPAD - _E,), _N, src.dtype)
    srcc = jnp.concatenate([src, padc])
    dstc = jnp.concatenate([dst, padc])

    zeros1 = jnp.zeros((_NPAD, 144), jnp.float32)
    zeros2 = jnp.zeros((_NPAD, 48), jnp.float32)
    zerosc = jnp.zeros((_NPAD, 16), jnp.float32)

    xp = _pad_rows(x)
    ep = _pad_rows(edge_type)
    node_repr = _gnn(xp, params['node'], srcp, dstp, zeros1, zeros2)
    edge_repr = _gnn(ep, params['edge'], srcp, dstp, zeros1, zeros2)

    c, pp = params['crf'], params['proxy']
    wf = jnp.concatenate([c['Wu'], pp['W1'], c['Wp']], axis=1)
    wf = jnp.pad(wf, ((0, 0), (0, 128 - wf.shape[1])))
    r = jnp.concatenate([node_repr, edge_repr], axis=0)
    o = _mm(r, wf)
    unary = o[:_N, 0:2] + c['bu']
    hpx = jax.nn.relu(o[:_N, 2:66] + pp['b1'])
    pair = o[_NPAD:_NPAD + _N, 66:68] + c['bp']

    q = jax.nn.softmax(unary, axis=-1)
    t = jnp.zeros((_NPAD, 16), jnp.float32).at[:_N, :2].set(q * pair)
    m0, m1 = _sc_crf(t, srcc, dstc, zerosc)
    crf_out = unary + (m0 + m1)[:_N, :2]

    w2 = jnp.pad(pp['W2'], ((0, 0), (0, 128 - _NCLS)))
    proxy_out = _mm(hpx, w2)[:, :2] + pp['b2']
    return (crf_out, proxy_out)


# speedup vs baseline: 2.3332x; 2.3332x over previous
import functools

import jax
import jax.numpy as jnp
from jax import lax
from jax.experimental import pallas as pl
from jax.experimental.pallas import tpu as pltpu
from jax.experimental.pallas import tpu_sc as plsc

_N = 10000
_E = 160000
_F = 128
_HID = 32
_HEADS = 8
_NCLS = 2
_PHID = 64

_NSUB = 16
_NCORE = 2
_BE1 = 48
_BEC = 64
_NPAD = 10112
_ROWS_PER_TILE = _NPAD // _NSUB
_EG = _E + _N
_E1PAD = 172032
_ECPAD = 163840


def _mm_body(x_ref, w_ref, o_ref):
    o_ref[...] = jnp.dot(x_ref[...], w_ref[...],
                         preferred_element_type=jnp.float32)


def _mm(x, w, bm=1024):
    m, k = x.shape
    n = w.shape[1]
    return pl.pallas_call(
        _mm_body,
        grid=(pl.cdiv(m, bm),),
        in_specs=[pl.BlockSpec((bm, k), lambda i: (i, 0)),
                  pl.BlockSpec((k, n), lambda i: (0, 0))],
        out_specs=pl.BlockSpec((bm, n), lambda i: (i, 0)),
        out_shape=jax.ShapeDtypeStruct((m, n), jnp.float32),
    )(x, w)


def _gat_chunks(src_hbm, dst_hbm, xl_hbm, xr_hbm, att_ref, accS, bufs,
                tile_base, nchunks, hpc, chans, be):
    w = hpc * chans
    nv = chans // 16
    srcv, dstv, dsts, xlb, xrb, mzb, sgl, sgr, ss, si = bufs
    iota16 = lax.iota(jnp.int32, 16)
    perms = [(iota16 ^ s).reshape(16, 1) for s in (8, 4, 2, 1)]
    gdn = lax.GatherDimensionNumbers(
        offset_dims=(), collapsed_slice_dims=(0,), start_index_map=(0,))

    def _lane_shuffle(v, p):
        return lax.gather(v, p, gdn, slice_sizes=(1,),
                          mode=lax.GatherScatterMode.PROMISE_IN_BOUNDS)

    att_vecs = [att_ref[pl.ds(k * 16, 16)] for k in range(w // 16)]

    def make_edge_body(b):
        def edge_body(e):
            zvec = jnp.zeros((16,), jnp.float32)
            for h in range(hpc):
                xs = []
                tot = None
                for k in range(nv):
                    off = h * chans + k * 16
                    xv = xlb[b][e, pl.ds(off, 16)]
                    rv = xrb[b][e, pl.ds(off, 16)]
                    t = xv + rv
                    t = jnp.where(t >= 0, t, t * jnp.float32(0.2))
                    t = t * att_vecs[off // 16]
                    xs.append(xv)
                    tot = t if tot is None else tot + t
                for p in perms:
                    tot = tot + _lane_shuffle(tot, p)
                sv = jnp.exp(tot)
                for k in range(nv):
                    off = h * chans + k * 16
                    mzb[b][e, pl.ds(off, 16)] = sv * xs[k]
                zvec = jnp.where(iota16 == h, sv, zvec)
            mzb[b][e, pl.ds(w, 16)] = zvec
        return edge_body

    def idx_fetch_async(j, b):
        base = tile_base + j * be
        pltpu.async_copy(src_hbm.at[pl.ds(base, be)], srcv[b], si[b])
        pltpu.async_copy(dst_hbm.at[pl.ds(base, be)], dstv[b], si[b])

    def idx_wait(j, b):
        base = tile_base + j * be
        pltpu.make_async_copy(src_hbm.at[pl.ds(base, be)], srcv[b],
                              si[b]).wait()
        pltpu.make_async_copy(dst_hbm.at[pl.ds(base, be)], dstv[b],
                              si[b]).wait()

    def gather_start(b):
        pltpu.async_copy(xl_hbm.at[srcv[b]], xlb[b], sgl[b])
        pltpu.async_copy(xr_hbm.at[dstv[b]], xrb[b], sgr[b])

    base0 = tile_base
    pltpu.sync_copy(src_hbm.at[pl.ds(base0, be)], srcv[0])
    pltpu.sync_copy(dst_hbm.at[pl.ds(base0, be)], dstv[0])
    gather_start(0)
    idx_fetch_async(1, 1)

    def pair_body(jj, carry):
        for b in range(2):
            j = jj * 2 + b
            @pl.when(j >= 1)
            def _():
                pltpu.make_async_copy(mzb[1 - b], accS.at[dsts[1 - b]],
                                      ss[1 - b]).wait()

            @pl.when(j + 1 < nchunks)
            def _():
                idx_wait(j + 1, 1 - b)
                gather_start(1 - b)

            pltpu.make_async_copy(xl_hbm.at[srcv[b]], xlb[b], sgl[b]).wait()
            pltpu.make_async_copy(xr_hbm.at[dstv[b]], xrb[b], sgr[b]).wait()
            for k in range(be // 16):
                dsts[b][pl.ds(k * 16, 16)] = dstv[b][pl.ds(k * 16, 16)]

            @pl.when(j + 2 < nchunks)
            def _():
                idx_fetch_async(j + 2, b)

            plsc.parallel_loop(0, be, 1, unroll=4)(make_edge_body(b))
            pltpu.async_copy(mzb[b], accS.at[dsts[b]], ss[b], add=True)
        return carry

    lax.fori_loop(0, nchunks // 2, pair_body, 0)
    pltpu.make_async_copy(mzb[1], accS.at[dsts[1]], ss[1]).wait()


def _gat_core(src_hbm, dst_hbm, xl_hbm, xr_hbm, att_hbm, zero_hbm, out_hbm,
              bufs, attv, accS, sid, tile_base, nchunks, hpc, chans, be):
    r0 = sid * _ROWS_PER_TILE
    pltpu.sync_copy(zero_hbm.at[pl.ds(r0, _ROWS_PER_TILE)],
                    accS.at[pl.ds(r0, _ROWS_PER_TILE)])
    pltpu.sync_copy(att_hbm, attv)
    plsc.subcore_barrier()
    _gat_chunks(src_hbm, dst_hbm, xl_hbm, xr_hbm, attv, accS, bufs,
                tile_base, nchunks, hpc, chans, be)
    plsc.subcore_barrier()
    pltpu.sync_copy(accS.at[pl.ds(r0, _ROWS_PER_TILE)],
                    out_hbm.at[pl.ds(r0, _ROWS_PER_TILE)])


def _gat_scratch(w):
    dbl = lambda t: [t, t]
    return (dbl(pltpu.VMEM((_BE1,), jnp.int32)) +
            dbl(pltpu.VMEM((_BE1,), jnp.int32)) +
            dbl(pltpu.VMEM((_BE1,), jnp.int32)) +
            dbl(pltpu.VMEM((_BE1, w), jnp.float32)) +
            dbl(pltpu.VMEM((_BE1, w), jnp.float32)) +
            dbl(pltpu.VMEM((_BE1, w + 16), jnp.float32)) +
            [pltpu.VMEM((w,), jnp.float32),
             pltpu.VMEM_SHARED((_NPAD, w + 16), jnp.float32)] +
            [pltpu.SemaphoreType.DMA] * 8)


def _pack_bufs(args):
    (s0, s1, d0, d1, e0, e1, xl0, xl1, xr0, xr1, m0, m1,
     attv, accS, g0, g1, g2, g3, ss0, ss1, si0, si1) = args
    bufs = ([s0, s1], [d0, d1], [e0, e1], [xl0, xl1], [xr0, xr1], [m0, m1],
            [g0, g1], [g2, g3], [ss0, ss1], [si0, si1])
    return bufs, attv, accS


def _sc_gat1(xl0, xl1, xr0, xr1, att0, att1, srcp, dstp, zeros):
    hpc, chans = _HEADS // _NCORE, _HID
    w = hpc * chans
    nchunks = _E1PAD // _NSUB // _BE1
    mesh = plsc.VectorSubcoreMesh(core_axis_name="c", subcore_axis_name="s")

    @functools.partial(
        pl.kernel,
        out_type=[jax.ShapeDtypeStruct((_NPAD, w + 16), jnp.float32),
                  jax.ShapeDtypeStruct((_NPAD, w + 16), jnp.float32)],
        mesh=mesh,
        compiler_params=pltpu.CompilerParams(use_tc_tiling_on_sc=False,
                                             needs_layout_passes=False),
        scratch_types=_gat_scratch(w),
    )
    def k(src_hbm, dst_hbm, xl0_h, xl1_h, xr0_h, xr1_h, a0_h, a1_h, z_h,
          out0, out1, *scratch):
        bufs, attv, accS = _pack_bufs(scratch)
        cid = lax.axis_index("c")
        sid = lax.axis_index("s")
        tile_base = sid * (nchunks * _BE1)

        @pl.when(cid == 0)
        def _():
            _gat_core(src_hbm, dst_hbm, xl0_h, xr0_h, a0_h, z_h, out0,
                      bufs, attv, accS, sid, tile_base, nchunks, hpc, chans,
                      _BE1)

        @pl.when(cid == 1)
        def _():
            _gat_core(src_hbm, dst_hbm, xl1_h, xr1_h, a1_h, z_h, out1,
                      bufs, attv, accS, sid, tile_base, nchunks, hpc, chans,
                      _BE1)

    return k(srcp, dstp, xl0, xl1, xr0, xr1, att0, att1, zeros)


def _sc_gat2(xl, xr, att, srcp, dstp, zeros):
    hpc, chans = 1, _HID
    w = hpc * chans
    nchunks = _E1PAD // (_NSUB * _NCORE) // _BE1
    mesh = plsc.VectorSubcoreMesh(core_axis_name="c", subcore_axis_name="s")

    @functools.partial(
        pl.kernel,
        out_type=[jax.ShapeDtypeStruct((_NPAD, w + 16), jnp.float32),
                  jax.ShapeDtypeStruct((_NPAD, w + 16), jnp.float32)],
        mesh=mesh,
        compiler_params=pltpu.CompilerParams(use_tc_tiling_on_sc=False,
                                             needs_layout_passes=False),
        scratch_types=_gat_scratch(w),
    )
    def k(src_hbm, dst_hbm, xl_h, xr_h, a_h, z_h, out0, out1, *scratch):
        bufs, attv, accS = _pack_bufs(scratch)
        cid = lax.axis_index("c")
        sid = lax.axis_index("s")
        tile_base = (cid * _NSUB + sid) * (nchunks * _BE1)

        @pl.when(cid == 0)
        def _():
            _gat_core(src_hbm, dst_hbm, xl_h, xr_h, a_h, z_h, out0,
                      bufs, attv, accS, sid, tile_base, nchunks, hpc, chans,
                      _BE1)

        @pl.when(cid == 1)
        def _():
            _gat_core(src_hbm, dst_hbm, xl_h, xr_h, a_h, z_h, out1,
                      bufs, attv, accS, sid, tile_base, nchunks, hpc, chans,
                      _BE1)

    return k(srcp, dstp, xl, xr, att, zeros)


def _sc_crf(t, srcc, dstc, zeros):
    w = 16
    nchunks = _ECPAD // (_NSUB * _NCORE) // _BEC
    mesh = plsc.VectorSubcoreMesh(core_axis_name="c", subcore_axis_name="s")

    @functools.partial(
        pl.kernel,
        out_type=[jax.ShapeDtypeStruct((_NPAD, w), jnp.float32),
                  jax.ShapeDtypeStruct((_NPAD, w), jnp.float32)],
        mesh=mesh,
        compiler_params=pltpu.CompilerParams(use_tc_tiling_on_sc=False,
                                             needs_layout_passes=False),
        scratch_types=[
            pltpu.VMEM((_BEC,), jnp.int32), pltpu.VMEM((_BEC,), jnp.int32),
            pltpu.VMEM((_BEC,), jnp.int32), pltpu.VMEM((_BEC,), jnp.int32),
            pltpu.VMEM((_BEC, w), jnp.float32),
            pltpu.VMEM((_BEC, w), jnp.float32),
            pltpu.VMEM_SHARED((_NPAD, w), jnp.float32),
        ] + [pltpu.SemaphoreType.DMA] * 4,
    )
    def k(src_hbm, dst_hbm, t_h, z_h, out0, out1,
          s0, s1, d0, d1, b0, b1, accS, g0, g1, ss0, ss1):
        srcv, dstv, buf = [s0, s1], [d0, d1], [b0, b1]
        sg, ss = [g0, g1], [ss0, ss1]
        cid = lax.axis_index("c")
        sid = lax.axis_index("s")
        tile_base = (cid * _NSUB + sid) * (nchunks * _BEC)
        r0 = sid * _ROWS_PER_TILE

        def prefetch(j, b):
            base = tile_base + j * _BEC
            pltpu.sync_copy(src_hbm.at[pl.ds(base, _BEC)], srcv[b])
            pltpu.sync_copy(dst_hbm.at[pl.ds(base, _BEC)], dstv[b])
            pltpu.async_copy(t_h.at[srcv[b]], buf[b], sg[b])

        def core(out_hbm):
            pltpu.sync_copy(z_h.at[pl.ds(r0, _ROWS_PER_TILE)],
                            accS.at[pl.ds(r0, _ROWS_PER_TILE)])
            plsc.subcore_barrier()
            prefetch(0, 0)

            def pair_body(jj, carry):
                for b in range(2):
                    j = jj * 2 + b

                    @pl.when(j >= 1)
                    def _():
                        pltpu.make_async_copy(buf[1 - b],
                                              accS.at[dstv[1 - b]],
                                              ss[1 - b]).wait()

                    @pl.when(j + 1 < nchunks)
                    def _():
                        prefetch(j + 1, 1 - b)

                    pltpu.make_async_copy(t_h.at[srcv[b]], buf[b],
                                          sg[b]).wait()
                    pltpu.async_copy(buf[b], accS.at[dstv[b]], ss[b],
                                     add=True)
                return carry

            lax.fori_loop(0, nchunks // 2, pair_body, 0)
            pltpu.make_async_copy(buf[1], accS.at[dstv[1]], ss[1]).wait()
            plsc.subcore_barrier()
            pltpu.sync_copy(accS.at[pl.ds(r0, _ROWS_PER_TILE)],
                            out_hbm.at[pl.ds(r0, _ROWS_PER_TILE)])

        @pl.when(cid == 0)
        def _():
            core(out0)

        @pl.when(cid == 1)
        def _():
            core(out1)

    return k(srcc, dstc, t, zeros)


def _pad_rows(a):
    return jnp.pad(a, ((0, _NPAD - a.shape[0]), (0, 0)))


def _gat_layer1(proj, a1, b1, srcp, dstp, zeros1):
    xl0 = proj[:, 0:128]
    xl1 = proj[:, 128:256]
    xr0 = proj[:, 256:384]
    xr1 = proj[:, 384:512]
    att0 = a1[:4].reshape(-1)
    att1 = a1[4:].reshape(-1)
    o0, o1 = _sc_gat1(xl0, xl1, xr0, xr1, att0, att1, srcp, dstp, zeros1)
    acc = jnp.concatenate([o0[:, :128], o1[:, :128]], axis=1)
    z = jnp.concatenate([o0[:, 128:132], o1[:, 128:132]], axis=1)
    zr = jnp.repeat(z, _HID, axis=1)
    return jax.nn.elu(acc / (zr + 1e-16) + b1)


def _gat_layer2(proj, a2, b2, srcp, dstp, zeros2):
    xl = proj[:, :_HID]
    xr = proj[:, _HID:]
    o0, o1 = _sc_gat2(xl, xr, a2.reshape(-1), srcp, dstp, zeros2)
    s = o0 + o1
    return s[:, :_HID] / (s[:, _HID:_HID + 1] + 1e-16) + b2


def _gnn(xp, p, srcp, dstp, zeros1, zeros2):
    p1 = _mm(xp, jnp.concatenate([p['Wl1'], p['Wr1']], axis=1))
    h1 = _gat_layer1(p1, p['a1'], p['b1'], srcp, dstp, zeros1)
    p2 = _mm(h1, jnp.concatenate([p['Wl2'], p['Wr2']], axis=1))
    return _gat_layer2(p2, p['a2'], p['b2'], srcp, dstp, zeros2)


def kernel(x, edge_type, edge_index, params):
    src = edge_index[0]
    dst = edge_index[1]
    si = jnp.arange(_N, dtype=src.dtype)
    padv = jnp.full((_E1PAD - _EG,), _N, src.dtype)
    srcp = jnp.concatenate([src, si, padv])
    dstp = jnp.concatenate([dst, si, padv])
    padc = jnp.full((_ECPAD - _E,), _N, src.dtype)
    srcc = jnp.concatenate([src, padc])
    dstc = jnp.concatenate([dst, padc])

    zeros1 = jnp.zeros((_NPAD, 144), jnp.float32)
    zeros2 = jnp.zeros((_NPAD, 48), jnp.float32)
    zerosc = jnp.zeros((_NPAD, 16), jnp.float32)

    xp = _pad_rows(x)
    ep = _pad_rows(edge_type)
    node_repr = _gnn(xp, params['node'], srcp, dstp, zeros1, zeros2)
    edge_repr = _gnn(ep, params['edge'], srcp, dstp, zeros1, zeros2)

    c, pp = params['crf'], params['proxy']
    wf = jnp.concatenate([c['Wu'], pp['W1'], c['Wp']], axis=1)
    wf = jnp.pad(wf, ((0, 0), (0, 128 - wf.shape[1])))
    r = jnp.concatenate([node_repr, edge_repr], axis=0)
    o = _mm(r, wf)
    unary = o[:_N, 0:2] + c['bu']
    hpx = jax.nn.relu(o[:_N, 2:66] + pp['b1'])
    pair = o[_NPAD:_NPAD + _N, 66:68] + c['bp']

    q = jax.nn.softmax(unary, axis=-1)
    t = jnp.zeros((_NPAD, 16), jnp.float32).at[:_N, :2].set(q * pair)
    m0, m1 = _sc_crf(t, srcc, dstc, zerosc)
    crf_out = unary + (m0 + m1)[:_N, :2]

    w2 = jnp.pad(pp['W2'], ((0, 0), (0, 128 - _NCLS)))
    proxy_out = _mm(hpx, w2)[:, :2] + pp['b2']
    return (crf_out, proxy_out)


# speedup vs baseline: 2.3768x; 1.0187x over previous
import functools

import jax
import jax.numpy as jnp
from jax import lax
from jax.experimental import pallas as pl
from jax.experimental.pallas import tpu as pltpu
from jax.experimental.pallas import tpu_sc as plsc

_N = 10000
_E = 160000
_F = 128
_HID = 32
_HEADS = 8
_NCLS = 2
_PHID = 64

_NSUB = 16
_NCORE = 2
_BE1 = 48
_BEC = 64
_NPAD = 10112
_ROWS_PER_TILE = _NPAD // _NSUB
_EG = _E + _N
_E1PAD = 172032
_ECPAD = 163840


def _mm_body(x_ref, w_ref, o_ref):
    o_ref[...] = jnp.dot(x_ref[...], w_ref[...],
                         preferred_element_type=jnp.float32)


def _mm(x, w, bm=1024):
    m, k = x.shape
    n = w.shape[1]
    return pl.pallas_call(
        _mm_body,
        grid=(pl.cdiv(m, bm),),
        in_specs=[pl.BlockSpec((bm, k), lambda i: (i, 0)),
                  pl.BlockSpec((k, n), lambda i: (0, 0))],
        out_specs=pl.BlockSpec((bm, n), lambda i: (i, 0)),
        out_shape=jax.ShapeDtypeStruct((m, n), jnp.float32),
    )(x, w)


def _gat_chunks(src_hbm, dst_hbm, xl_hbm, xr_hbm, att_ref, accS, bufs,
                tile_base, nchunks, hpc, chans, be):
    w = hpc * chans
    nv = chans // 16
    srcv, dstv, dsts, xlb, xrb, mzb, sgl, sgr, ss, si = bufs
    iota16 = lax.iota(jnp.int32, 16)
    perms = [(iota16 ^ s).reshape(16, 1) for s in (8, 4, 2, 1)]
    gdn = lax.GatherDimensionNumbers(
        offset_dims=(), collapsed_slice_dims=(0,), start_index_map=(0,))

    def _lane_shuffle(v, p):
        return lax.gather(v, p, gdn, slice_sizes=(1,),
                          mode=lax.GatherScatterMode.PROMISE_IN_BOUNDS)

    att_vecs = [att_ref[pl.ds(k * 16, 16)] for k in range(w // 16)]

    def make_edge_body(b):
        def edge_body(e):
            zvec = jnp.zeros((16,), jnp.float32)
            for h in range(hpc):
                xs = []
                tot = None
                for k in range(nv):
                    off = h * chans + k * 16
                    xv = xlb[b][e, pl.ds(off, 16)]
                    rv = xrb[b][e, pl.ds(off, 16)]
                    t = xv + rv
                    t = jnp.where(t >= 0, t, t * jnp.float32(0.2))
                    t = t * att_vecs[off // 16]
                    xs.append(xv)
                    tot = t if tot is None else tot + t
                for p in perms:
                    tot = tot + _lane_shuffle(tot, p)
                sv = jnp.exp(tot)
                for k in range(nv):
                    off = h * chans + k * 16
                    mzb[b][e, pl.ds(off, 16)] = sv * xs[k]
                zvec = jnp.where(iota16 == h, sv, zvec)
            mzb[b][e, pl.ds(w, 16)] = zvec
        return edge_body

    def idx_fetch_async(j, b):
        base = tile_base + j * be
        pltpu.async_copy(src_hbm.at[pl.ds(base, be)], srcv[b], si[b])
        pltpu.async_copy(dst_hbm.at[pl.ds(base, be)], dstv[b], si[b])

    def idx_wait(j, b):
        base = tile_base + j * be
        pltpu.make_async_copy(src_hbm.at[pl.ds(base, be)], srcv[b],
                              si[b]).wait()
        pltpu.make_async_copy(dst_hbm.at[pl.ds(base, be)], dstv[b],
                              si[b]).wait()

    def gather_start(b):
        pltpu.async_copy(xl_hbm.at[srcv[b]], xlb[b], sgl[b])
        pltpu.async_copy(xr_hbm.at[dstv[b]], xrb[b], sgr[b])

    base0 = tile_base
    pltpu.sync_copy(src_hbm.at[pl.ds(base0, be)], srcv[0])
    pltpu.sync_copy(dst_hbm.at[pl.ds(base0, be)], dstv[0])
    gather_start(0)
    idx_fetch_async(1, 1)

    def pair_body(jj, carry):
        for b in range(2):
            j = jj * 2 + b
            @pl.when(j >= 1)
            def _():
                pltpu.make_async_copy(mzb[1 - b], accS.at[dsts[1 - b]],
                                      ss[1 - b]).wait()

            @pl.when(j + 1 < nchunks)
            def _():
                idx_wait(j + 1, 1 - b)
                gather_start(1 - b)

            pltpu.make_async_copy(xl_hbm.at[srcv[b]], xlb[b], sgl[b]).wait()
            pltpu.make_async_copy(xr_hbm.at[dstv[b]], xrb[b], sgr[b]).wait()
            for k in range(be // 16):
                dsts[b][pl.ds(k * 16, 16)] = dstv[b][pl.ds(k * 16, 16)]

            @pl.when(j + 2 < nchunks)
            def _():
                idx_fetch_async(j + 2, b)

            plsc.parallel_loop(0, be, 1, unroll=4)(make_edge_body(b))
            pltpu.async_copy(mzb[b], accS.at[dsts[b]], ss[b], add=True)
        return carry

    lax.fori_loop(0, nchunks // 2, pair_body, 0)
    pltpu.make_async_copy(mzb[1], accS.at[dsts[1]], ss[1]).wait()


def _gat_core(src_hbm, dst_hbm, xl_hbm, xr_hbm, att_hbm, zero_hbm, out_hbm,
              bufs, attv, accS, sid, tile_base, nchunks, hpc, chans, be):
    r0 = sid * _ROWS_PER_TILE
    pltpu.sync_copy(zero_hbm.at[pl.ds(r0, _ROWS_PER_TILE)],
                    accS.at[pl.ds(r0, _ROWS_PER_TILE)])
    pltpu.sync_copy(att_hbm, attv)
    plsc.subcore_barrier()
    _gat_chunks(src_hbm, dst_hbm, xl_hbm, xr_hbm, attv, accS, bufs,
                tile_base, nchunks, hpc, chans, be)
    plsc.subcore_barrier()
    pltpu.sync_copy(accS.at[pl.ds(r0, _ROWS_PER_TILE)],
                    out_hbm.at[pl.ds(r0, _ROWS_PER_TILE)])


def _gat_scratch(w):
    dbl = lambda t: [t, t]
    return (dbl(pltpu.VMEM((_BE1,), jnp.int32)) +
            dbl(pltpu.VMEM((_BE1,), jnp.int32)) +
            dbl(pltpu.VMEM((_BE1,), jnp.int32)) +
            dbl(pltpu.VMEM((_BE1, w), jnp.float32)) +
            dbl(pltpu.VMEM((_BE1, w), jnp.float32)) +
            dbl(pltpu.VMEM((_BE1, w + 16), jnp.float32)) +
            [pltpu.VMEM((w,), jnp.float32),
             pltpu.VMEM_SHARED((_NPAD, w + 16), jnp.float32)] +
            [pltpu.SemaphoreType.DMA] * 8)


def _pack_bufs(args):
    (s0, s1, d0, d1, e0, e1, xl0, xl1, xr0, xr1, m0, m1,
     attv, accS, g0, g1, g2, g3, ss0, ss1, si0, si1) = args
    bufs = ([s0, s1], [d0, d1], [e0, e1], [xl0, xl1], [xr0, xr1], [m0, m1],
            [g0, g1], [g2, g3], [ss0, ss1], [si0, si1])
    return bufs, attv, accS


def _sc_gat1(xl0, xl1, xr0, xr1, att0, att1, srcp, dstp, zeros):
    hpc, chans = _HEADS // _NCORE, _HID
    w = hpc * chans
    nchunks = _E1PAD // _NSUB // _BE1
    mesh = plsc.VectorSubcoreMesh(core_axis_name="c", subcore_axis_name="s")

    @functools.partial(
        pl.kernel,
        out_type=[jax.ShapeDtypeStruct((_NPAD, w + 16), jnp.float32),
                  jax.ShapeDtypeStruct((_NPAD, w + 16), jnp.float32)],
        mesh=mesh,
        compiler_params=pltpu.CompilerParams(use_tc_tiling_on_sc=False,
                                             needs_layout_passes=False),
        scratch_types=_gat_scratch(w),
    )
    def k(src_hbm, dst_hbm, xl0_h, xl1_h, xr0_h, xr1_h, a0_h, a1_h, z_h,
          out0, out1, *scratch):
        bufs, attv, accS = _pack_bufs(scratch)
        cid = lax.axis_index("c")
        sid = lax.axis_index("s")
        tile_base = sid * (nchunks * _BE1)

        @pl.when(cid == 0)
        def _():
            _gat_core(src_hbm, dst_hbm, xl0_h, xr0_h, a0_h, z_h, out0,
                      bufs, attv, accS, sid, tile_base, nchunks, hpc, chans,
                      _BE1)

        @pl.when(cid == 1)
        def _():
            _gat_core(src_hbm, dst_hbm, xl1_h, xr1_h, a1_h, z_h, out1,
                      bufs, attv, accS, sid, tile_base, nchunks, hpc, chans,
                      _BE1)

    return k(srcp, dstp, xl0, xl1, xr0, xr1, att0, att1, zeros)


def _sc_gat2(xl, xr, att, srcp, dstp, zeros):
    hpc, chans = 1, _HID
    w = hpc * chans
    nchunks = _E1PAD // (_NSUB * _NCORE) // _BE1
    mesh = plsc.VectorSubcoreMesh(core_axis_name="c", subcore_axis_name="s")

    @functools.partial(
        pl.kernel,
        out_type=[jax.ShapeDtypeStruct((_NPAD, w + 16), jnp.float32),
                  jax.ShapeDtypeStruct((_NPAD, w + 16), jnp.float32)],
        mesh=mesh,
        compiler_params=pltpu.CompilerParams(use_tc_tiling_on_sc=False,
                                             needs_layout_passes=False),
        scratch_types=_gat_scratch(w),
    )
    def k(src_hbm, dst_hbm, xl_h, xr_h, a_h, z_h, out0, out1, *scratch):
        bufs, attv, accS = _pack_bufs(scratch)
        cid = lax.axis_index("c")
        sid = lax.axis_index("s")
        tile_base = (cid * _NSUB + sid) * (nchunks * _BE1)

        @pl.when(cid == 0)
        def _():
            _gat_core(src_hbm, dst_hbm, xl_h, xr_h, a_h, z_h, out0,
                      bufs, attv, accS, sid, tile_base, nchunks, hpc, chans,
                      _BE1)

        @pl.when(cid == 1)
        def _():
            _gat_core(src_hbm, dst_hbm, xl_h, xr_h, a_h, z_h, out1,
                      bufs, attv, accS, sid, tile_base, nchunks, hpc, chans,
                      _BE1)

    return k(srcp, dstp, xl, xr, att, zeros)


def _sc_crf(t, srcc, dstc, zeros):
    w = 16
    nchunks = _ECPAD // (_NSUB * _NCORE) // _BEC
    mesh = plsc.VectorSubcoreMesh(core_axis_name="c", subcore_axis_name="s")

    @functools.partial(
        pl.kernel,
        out_type=[jax.ShapeDtypeStruct((_NPAD, w), jnp.float32),
                  jax.ShapeDtypeStruct((_NPAD, w), jnp.float32)],
        mesh=mesh,
        compiler_params=pltpu.CompilerParams(use_tc_tiling_on_sc=False,
                                             needs_layout_passes=False),
        scratch_types=[
            pltpu.VMEM((_BEC,), jnp.int32), pltpu.VMEM((_BEC,), jnp.int32),
            pltpu.VMEM((_BEC,), jnp.int32), pltpu.VMEM((_BEC,), jnp.int32),
            pltpu.VMEM((_BEC,), jnp.int32), pltpu.VMEM((_BEC,), jnp.int32),
            pltpu.VMEM((_BEC, w), jnp.float32),
            pltpu.VMEM((_BEC, w), jnp.float32),
            pltpu.VMEM_SHARED((_NPAD, w), jnp.float32),
        ] + [pltpu.SemaphoreType.DMA] * 6,
    )
    def k(src_hbm, dst_hbm, t_h, z_h, out0, out1,
          s0, s1, d0, d1, e0, e1, b0, b1, accS, g0, g1, ss0, ss1, si0, si1):
        srcv, dstv, dsts, buf = [s0, s1], [d0, d1], [e0, e1], [b0, b1]
        sg, ss, si = [g0, g1], [ss0, ss1], [si0, si1]
        cid = lax.axis_index("c")
        sid = lax.axis_index("s")
        tile_base = (cid * _NSUB + sid) * (nchunks * _BEC)
        r0 = sid * _ROWS_PER_TILE

        def idx_fetch_async(j, b):
            base = tile_base + j * _BEC
            pltpu.async_copy(src_hbm.at[pl.ds(base, _BEC)], srcv[b], si[b])
            pltpu.async_copy(dst_hbm.at[pl.ds(base, _BEC)], dstv[b], si[b])

        def idx_wait(j, b):
            base = tile_base + j * _BEC
            pltpu.make_async_copy(src_hbm.at[pl.ds(base, _BEC)], srcv[b],
                                  si[b]).wait()
            pltpu.make_async_copy(dst_hbm.at[pl.ds(base, _BEC)], dstv[b],
                                  si[b]).wait()

        def core(out_hbm):
            pltpu.sync_copy(z_h.at[pl.ds(r0, _ROWS_PER_TILE)],
                            accS.at[pl.ds(r0, _ROWS_PER_TILE)])
            plsc.subcore_barrier()
            pltpu.sync_copy(src_hbm.at[pl.ds(tile_base, _BEC)], srcv[0])
            pltpu.sync_copy(dst_hbm.at[pl.ds(tile_base, _BEC)], dstv[0])
            pltpu.async_copy(t_h.at[srcv[0]], buf[0], sg[0])
            idx_fetch_async(1, 1)

            def pair_body(jj, carry):
                for b in range(2):
                    j = jj * 2 + b

                    @pl.when(j >= 1)
                    def _():
                        pltpu.make_async_copy(buf[1 - b],
                                              accS.at[dsts[1 - b]],
                                              ss[1 - b]).wait()

                    @pl.when(j + 1 < nchunks)
                    def _():
                        idx_wait(j + 1, 1 - b)
                        pltpu.async_copy(t_h.at[srcv[1 - b]], buf[1 - b],
                                         sg[1 - b])

                    pltpu.make_async_copy(t_h.at[srcv[b]], buf[b],
                                          sg[b]).wait()
                    for k2 in range(_BEC // 16):
                        dsts[b][pl.ds(k2 * 16, 16)] = \
                            dstv[b][pl.ds(k2 * 16, 16)]

                    @pl.when(j + 2 < nchunks)
                    def _():
                        idx_fetch_async(j + 2, b)

                    pltpu.async_copy(buf[b], accS.at[dsts[b]], ss[b],
                                     add=True)
                return carry

            lax.fori_loop(0, nchunks // 2, pair_body, 0)
            pltpu.make_async_copy(buf[1], accS.at[dsts[1]], ss[1]).wait()
            plsc.subcore_barrier()
            pltpu.sync_copy(accS.at[pl.ds(r0, _ROWS_PER_TILE)],
                            out_hbm.at[pl.ds(r0, _ROWS_PER_TILE)])

        @pl.when(cid == 0)
        def _():
            core(out0)

        @pl.when(cid == 1)
        def _():
            core(out1)

    return k(srcc, dstc, t, zeros)


def _pad_rows(a):
    return jnp.pad(a, ((0, _NPAD - a.shape[0]), (0, 0)))


def _gat_layer1(proj, a1, b1, srcp, dstp, zeros1):
    xl0 = proj[:, 0:128]
    xl1 = proj[:, 128:256]
    xr0 = proj[:, 256:384]
    xr1 = proj[:, 384:512]
    att0 = a1[:4].reshape(-1)
    att1 = a1[4:].reshape(-1)
    o0, o1 = _sc_gat1(xl0, xl1, xr0, xr1, att0, att1, srcp, dstp, zeros1)
    acc = jnp.concatenate([o0[:, :128], o1[:, :128]], axis=1)
    z = jnp.concatenate([o0[:, 128:132], o1[:, 128:132]], axis=1)
    zr = jnp.repeat(z, _HID, axis=1)
    return jax.nn.elu(acc / (zr + 1e-16) + b1)


def _gat_layer2(proj, a2, b2, srcp, dstp, zeros2):
    xl = proj[:, :_HID]
    xr = proj[:, _HID:]
    o0, o1 = _sc_gat2(xl, xr, a2.reshape(-1), srcp, dstp, zeros2)
    s = o0 + o1
    return s[:, :_HID] / (s[:, _HID:_HID + 1] + 1e-16) + b2


def _gnn(xp, p, srcp, dstp, zeros1, zeros2):
    p1 = _mm(xp, jnp.concatenate([p['Wl1'], p['Wr1']], axis=1))
    h1 = _gat_layer1(p1, p['a1'], p['b1'], srcp, dstp, zeros1)
    p2 = _mm(h1, jnp.concatenate([p['Wl2'], p['Wr2']], axis=1))
    return _gat_layer2(p2, p['a2'], p['b2'], srcp, dstp, zeros2)


def kernel(x, edge_type, edge_index, params):
    src = edge_index[0]
    dst = edge_index[1]
    si = jnp.arange(_N, dtype=src.dtype)
    padv = jnp.full((_E1PAD - _EG,), _N, src.dtype)
    srcp = jnp.concatenate([src, si, padv])
    dstp = jnp.concatenate([dst, si, padv])
    padc = jnp.full((_ECPAD - _E,), _N, src.dtype)
    srcc = jnp.concatenate([src, padc])
    dstc = jnp.concatenate([dst, padc])

    zeros1 = jnp.zeros((_NPAD, 144), jnp.float32)
    zeros2 = jnp.zeros((_NPAD, 48), jnp.float32)
    zerosc = jnp.zeros((_NPAD, 16), jnp.float32)

    xp = _pad_rows(x)
    ep = _pad_rows(edge_type)
    node_repr = _gnn(xp, params['node'], srcp, dstp, zeros1, zeros2)
    edge_repr = _gnn(ep, params['edge'], srcp, dstp, zeros1, zeros2)

    c, pp = params['crf'], params['proxy']
    wf = jnp.concatenate([c['Wu'], pp['W1'], c['Wp']], axis=1)
    wf = jnp.pad(wf, ((0, 0), (0, 128 - wf.shape[1])))
    r = jnp.concatenate([node_repr, edge_repr], axis=0)
    o = _mm(r, wf)
    unary = o[:_N, 0:2] + c['bu']
    hpx = jax.nn.relu(o[:_N, 2:66] + pp['b1'])
    pair = o[_NPAD:_NPAD + _N, 66:68] + c['bp']

    q = jax.nn.softmax(unary, axis=-1)
    t = jnp.zeros((_NPAD, 16), jnp.float32).at[:_N, :2].set(q * pair)
    m0, m1 = _sc_crf(t, srcc, dstc, zerosc)
    crf_out = unary + (m0 + m1)[:_N, :2]

    w2 = jnp.pad(pp['W2'], ((0, 0), (0, 128 - _NCLS)))
    proxy_out = _mm(hpx, w2)[:, :2] + pp['b2']
    return (crf_out, proxy_out)


# speedup vs baseline: 2.4319x; 1.0232x over previous
import functools

import jax
import jax.numpy as jnp
from jax import lax
from jax.experimental import pallas as pl
from jax.experimental.pallas import tpu as pltpu
from jax.experimental.pallas import tpu_sc as plsc

_N = 10000
_E = 160000
_F = 128
_HID = 32
_HEADS = 8
_NCLS = 2
_PHID = 64

_NSUB = 16
_NCORE = 2
_BE1 = 48
_BEC = 64
_NPAD = 10112
_ROWS_PER_TILE = _NPAD // _NSUB
_EG = _E + _N
_E1PAD = 172032
_ECPAD = 163840


def _mm_body(x_ref, w_ref, o_ref):
    o_ref[...] = jnp.dot(x_ref[...], w_ref[...],
                         preferred_element_type=jnp.float32)


def _mm(x, w, bm=1024):
    m, k = x.shape
    n = w.shape[1]
    return pl.pallas_call(
        _mm_body,
        grid=(pl.cdiv(m, bm),),
        in_specs=[pl.BlockSpec((bm, k), lambda i: (i, 0)),
                  pl.BlockSpec((k, n), lambda i: (0, 0))],
        out_specs=pl.BlockSpec((bm, n), lambda i: (i, 0)),
        out_shape=jax.ShapeDtypeStruct((m, n), jnp.float32),
    )(x, w)


def _gat_chunks(src_hbm, dst_hbm, xl_hbm, xr_hbm, att_ref, accS, bufs,
                tile_base, nchunks, hpc, chans, be, dshift):
    w = hpc * chans
    nv = chans // 16
    srcv, dstv, dsts, xlb, xrb, mzb, sgl, sgr, ss, si = bufs
    iota16 = lax.iota(jnp.int32, 16)
    perms = [(iota16 ^ s).reshape(16, 1) for s in (8, 4, 2, 1)]
    gdn = lax.GatherDimensionNumbers(
        offset_dims=(), collapsed_slice_dims=(0,), start_index_map=(0,))

    def _lane_shuffle(v, p):
        return lax.gather(v, p, gdn, slice_sizes=(1,),
                          mode=lax.GatherScatterMode.PROMISE_IN_BOUNDS)

    att_vecs = [att_ref[pl.ds(k * 16, 16)] for k in range(w // 16)]

    def make_edge_body(b):
        def edge_body(e):
            zvec = jnp.zeros((16,), jnp.float32)
            for h in range(hpc):
                xs = []
                tot = None
                for k in range(nv):
                    off = h * chans + k * 16
                    xv = xlb[b][e, pl.ds(off, 16)]
                    rv = xrb[b][e, pl.ds(off, 16)]
                    t = xv + rv
                    t = jnp.where(t >= 0, t, t * jnp.float32(0.2))
                    t = t * att_vecs[off // 16]
                    xs.append(xv)
                    tot = t if tot is None else tot + t
                for p in perms:
                    tot = tot + _lane_shuffle(tot, p)
                sv = jnp.exp(tot)
                for k in range(nv):
                    off = h * chans + k * 16
                    mzb[b][e, pl.ds(off, 16)] = sv * xs[k]
                zvec = jnp.where(iota16 == h, sv, zvec)
            mzb[b][e, pl.ds(w, 16)] = zvec
        return edge_body

    def idx_fetch_async(j, b):
        base = tile_base + j * be
        pltpu.async_copy(src_hbm.at[pl.ds(base, be)], srcv[b], si[b])
        pltpu.async_copy(dst_hbm.at[pl.ds(base, be)], dstv[b], si[b])

    def idx_wait(j, b):
        base = tile_base + j * be
        pltpu.make_async_copy(src_hbm.at[pl.ds(base, be)], srcv[b],
                              si[b]).wait()
        pltpu.make_async_copy(dst_hbm.at[pl.ds(base, be)], dstv[b],
                              si[b]).wait()

    def gather_start(b):
        pltpu.async_copy(xl_hbm.at[srcv[b]], xlb[b], sgl[b])
        pltpu.async_copy(xr_hbm.at[dstv[b]], xrb[b], sgr[b])

    base0 = tile_base
    pltpu.sync_copy(src_hbm.at[pl.ds(base0, be)], srcv[0])
    pltpu.sync_copy(dst_hbm.at[pl.ds(base0, be)], dstv[0])
    gather_start(0)
    idx_fetch_async(1, 1)

    def pair_body(jj, carry):
        for b in range(2):
            j = jj * 2 + b
            @pl.when(j >= 1)
            def _():
                pltpu.make_async_copy(mzb[1 - b], accS.at[dsts[1 - b]],
                                      ss[1 - b]).wait()

            @pl.when(j + 1 < nchunks)
            def _():
                idx_wait(j + 1, 1 - b)
                gather_start(1 - b)

            pltpu.make_async_copy(xl_hbm.at[srcv[b]], xlb[b], sgl[b]).wait()
            pltpu.make_async_copy(xr_hbm.at[dstv[b]], xrb[b], sgr[b]).wait()
            for k in range(be // 16):
                val = dstv[b][pl.ds(k * 16, 16)]
                if dshift:
                    val = lax.shift_right_logical(val, jnp.int32(dshift))
                dsts[b][pl.ds(k * 16, 16)] = val

            @pl.when(j + 2 < nchunks)
            def _():
                idx_fetch_async(j + 2, b)

            plsc.parallel_loop(0, be, 1, unroll=4)(make_edge_body(b))
            pltpu.async_copy(mzb[b], accS.at[dsts[b]], ss[b], add=True)
        return carry

    lax.fori_loop(0, nchunks // 2, pair_body, 0)
    pltpu.make_async_copy(mzb[1], accS.at[dsts[1]], ss[1]).wait()


def _gat_core(src_hbm, dst_hbm, xl_hbm, xr_hbm, att_hbm, zero_hbm, out_hbm,
              bufs, attv, accS, sid, tile_base, nchunks, hpc, chans, be,
              dshift):
    r0 = sid * _ROWS_PER_TILE
    pltpu.sync_copy(zero_hbm.at[pl.ds(r0, _ROWS_PER_TILE)],
                    accS.at[pl.ds(r0, _ROWS_PER_TILE)])
    pltpu.sync_copy(att_hbm, attv)
    plsc.subcore_barrier()
    _gat_chunks(src_hbm, dst_hbm, xl_hbm, xr_hbm, attv, accS, bufs,
                tile_base, nchunks, hpc, chans, be, dshift)
    plsc.subcore_barrier()
    pltpu.sync_copy(accS.at[pl.ds(r0, _ROWS_PER_TILE)],
                    out_hbm.at[pl.ds(r0, _ROWS_PER_TILE)])


def _gat_scratch(w):
    dbl = lambda t: [t, t]
    return (dbl(pltpu.VMEM((_BE1,), jnp.int32)) +
            dbl(pltpu.VMEM((_BE1,), jnp.int32)) +
            dbl(pltpu.VMEM((_BE1,), jnp.int32)) +
            dbl(pltpu.VMEM((_BE1, w), jnp.float32)) +
            dbl(pltpu.VMEM((_BE1, w), jnp.float32)) +
            dbl(pltpu.VMEM((_BE1, w + 16), jnp.float32)) +
            [pltpu.VMEM((w,), jnp.float32),
             pltpu.VMEM_SHARED((_NPAD, w + 16), jnp.float32)] +
            [pltpu.SemaphoreType.DMA] * 8)


def _pack_bufs(args):
    (s0, s1, d0, d1, e0, e1, xl0, xl1, xr0, xr1, m0, m1,
     attv, accS, g0, g1, g2, g3, ss0, ss1, si0, si1) = args
    bufs = ([s0, s1], [d0, d1], [e0, e1], [xl0, xl1], [xr0, xr1], [m0, m1],
            [g0, g1], [g2, g3], [ss0, ss1], [si0, si1])
    return bufs, attv, accS


def _sc_gat1(table, att0, att1, src4, dst4, zeros):
    hpc, chans = _HEADS // _NCORE, _HID
    w = hpc * chans
    nchunks = _E1PAD // _NSUB // _BE1
    mesh = plsc.VectorSubcoreMesh(core_axis_name="c", subcore_axis_name="s")

    @functools.partial(
        pl.kernel,
        out_type=[jax.ShapeDtypeStruct((_NPAD, w + 16), jnp.float32),
                  jax.ShapeDtypeStruct((_NPAD, w + 16), jnp.float32)],
        mesh=mesh,
        compiler_params=pltpu.CompilerParams(use_tc_tiling_on_sc=False,
                                             needs_layout_passes=False),
        scratch_types=_gat_scratch(w),
    )
    def k(src0_h, src1_h, dst0_h, dst1_h, tab_h, a0_h, a1_h, z_h,
          out0, out1, *scratch):
        bufs, attv, accS = _pack_bufs(scratch)
        cid = lax.axis_index("c")
        sid = lax.axis_index("s")
        tile_base = sid * (nchunks * _BE1)

        @pl.when(cid == 0)
        def _():
            _gat_core(src0_h, dst0_h, tab_h, tab_h, a0_h, z_h, out0,
                      bufs, attv, accS, sid, tile_base, nchunks, hpc, chans,
                      _BE1, 2)

        @pl.when(cid == 1)
        def _():
            _gat_core(src1_h, dst1_h, tab_h, tab_h, a1_h, z_h, out1,
                      bufs, attv, accS, sid, tile_base, nchunks, hpc, chans,
                      _BE1, 2)

    return k(src4[0], src4[1], dst4[0], dst4[1], table, att0, att1, zeros)


def _sc_gat2(table, att, src2, dst2, zeros):
    hpc, chans = 1, _HID
    w = hpc * chans
    nchunks = _E1PAD // (_NSUB * _NCORE) // _BE1
    mesh = plsc.VectorSubcoreMesh(core_axis_name="c", subcore_axis_name="s")

    @functools.partial(
        pl.kernel,
        out_type=[jax.ShapeDtypeStruct((_NPAD, w + 16), jnp.float32),
                  jax.ShapeDtypeStruct((_NPAD, w + 16), jnp.float32)],
        mesh=mesh,
        compiler_params=pltpu.CompilerParams(use_tc_tiling_on_sc=False,
                                             needs_layout_passes=False),
        scratch_types=_gat_scratch(w),
    )
    def k(src_hbm, dst_hbm, tab_h, a_h, z_h, out0, out1, *scratch):
        bufs, attv, accS = _pack_bufs(scratch)
        cid = lax.axis_index("c")
        sid = lax.axis_index("s")
        tile_base = (cid * _NSUB + sid) * (nchunks * _BE1)

        @pl.when(cid == 0)
        def _():
            _gat_core(src_hbm, dst_hbm, tab_h, tab_h, a_h, z_h, out0,
                      bufs, attv, accS, sid, tile_base, nchunks, hpc, chans,
                      _BE1, 1)

        @pl.when(cid == 1)
        def _():
            _gat_core(src_hbm, dst_hbm, tab_h, tab_h, a_h, z_h, out1,
                      bufs, attv, accS, sid, tile_base, nchunks, hpc, chans,
                      _BE1, 1)

    return k(src2, dst2, table, att, zeros)


def _sc_crf(t, srcc, dstc, zeros):
    w = 16
    nchunks = _ECPAD // (_NSUB * _NCORE) // _BEC
    mesh = plsc.VectorSubcoreMesh(core_axis_name="c", subcore_axis_name="s")

    @functools.partial(
        pl.kernel,
        out_type=[jax.ShapeDtypeStruct((_NPAD, w), jnp.float32),
                  jax.ShapeDtypeStruct((_NPAD, w), jnp.float32)],
        mesh=mesh,
        compiler_params=pltpu.CompilerParams(use_tc_tiling_on_sc=False,
                                             needs_layout_passes=False),
        scratch_types=[
            pltpu.VMEM((_BEC,), jnp.int32), pltpu.VMEM((_BEC,), jnp.int32),
            pltpu.VMEM((_BEC,), jnp.int32), pltpu.VMEM((_BEC,), jnp.int32),
            pltpu.VMEM((_BEC,), jnp.int32), pltpu.VMEM((_BEC,), jnp.int32),
            pltpu.VMEM((_BEC, w), jnp.float32),
            pltpu.VMEM((_BEC, w), jnp.float32),
            pltpu.VMEM_SHARED((_NPAD, w), jnp.float32),
        ] + [pltpu.SemaphoreType.DMA] * 6,
    )
    def k(src_hbm, dst_hbm, t_h, z_h, out0, out1,
          s0, s1, d0, d1, e0, e1, b0, b1, accS, g0, g1, ss0, ss1, si0, si1):
        srcv, dstv, dsts, buf = [s0, s1], [d0, d1], [e0, e1], [b0, b1]
        sg, ss, si = [g0, g1], [ss0, ss1], [si0, si1]
        cid = lax.axis_index("c")
        sid = lax.axis_index("s")
        tile_base = (cid * _NSUB + sid) * (nchunks * _BEC)
        r0 = sid * _ROWS_PER_TILE

        def idx_fetch_async(j, b):
            base = tile_base + j * _BEC
            pltpu.async_copy(src_hbm.at[pl.ds(base, _BEC)], srcv[b], si[b])
            pltpu.async_copy(dst_hbm.at[pl.ds(base, _BEC)], dstv[b], si[b])

        def idx_wait(j, b):
            base = tile_base + j * _BEC
            pltpu.make_async_copy(src_hbm.at[pl.ds(base, _BEC)], srcv[b],
                                  si[b]).wait()
            pltpu.make_async_copy(dst_hbm.at[pl.ds(base, _BEC)], dstv[b],
                                  si[b]).wait()

        def core(out_hbm):
            pltpu.sync_copy(z_h.at[pl.ds(r0, _ROWS_PER_TILE)],
                            accS.at[pl.ds(r0, _ROWS_PER_TILE)])
            plsc.subcore_barrier()
            pltpu.sync_copy(src_hbm.at[pl.ds(tile_base, _BEC)], srcv[0])
            pltpu.sync_copy(dst_hbm.at[pl.ds(tile_base, _BEC)], dstv[0])
            pltpu.async_copy(t_h.at[srcv[0]], buf[0], sg[0])
            idx_fetch_async(1, 1)

            def pair_body(jj, carry):
                for b in range(2):
                    j = jj * 2 + b

                    @pl.when(j >= 1)
                    def _():
                        pltpu.make_async_copy(buf[1 - b],
                                              accS.at[dsts[1 - b]],
                                              ss[1 - b]).wait()

                    @pl.when(j + 1 < nchunks)
                    def _():
                        idx_wait(j + 1, 1 - b)
                        pltpu.async_copy(t_h.at[srcv[1 - b]], buf[1 - b],
                                         sg[1 - b])

                    pltpu.make_async_copy(t_h.at[srcv[b]], buf[b],
                                          sg[b]).wait()
                    for k2 in range(_BEC // 16):
                        dsts[b][pl.ds(k2 * 16, 16)] = \
                            dstv[b][pl.ds(k2 * 16, 16)]

                    @pl.when(j + 2 < nchunks)
                    def _():
                        idx_fetch_async(j + 2, b)

                    pltpu.async_copy(buf[b], accS.at[dsts[b]], ss[b],
                                     add=True)
                return carry

            lax.fori_loop(0, nchunks // 2, pair_body, 0)
            pltpu.make_async_copy(buf[1], accS.at[dsts[1]], ss[1]).wait()
            plsc.subcore_barrier()
            pltpu.sync_copy(accS.at[pl.ds(r0, _ROWS_PER_TILE)],
                            out_hbm.at[pl.ds(r0, _ROWS_PER_TILE)])

        @pl.when(cid == 0)
        def _():
            core(out0)

        @pl.when(cid == 1)
        def _():
            core(out1)

    return k(srcc, dstc, t, zeros)


def _pad_rows(a):
    return jnp.pad(a, ((0, _NPAD - a.shape[0]), (0, 0)))


def _gat_layer1(proj, a1, b1, src4, dst4, zeros1):
    table = proj.reshape(-1, 128)
    att0 = a1[:4].reshape(-1)
    att1 = a1[4:].reshape(-1)
    o0, o1 = _sc_gat1(table, att0, att1, src4, dst4, zeros1)
    acc = jnp.concatenate([o0[:, :128], o1[:, :128]], axis=1)
    z = jnp.concatenate([o0[:, 128:132], o1[:, 128:132]], axis=1)
    zr = jnp.repeat(z, _HID, axis=1)
    return jax.nn.elu(acc / (zr + 1e-16) + b1)


def _gat_layer2(proj, a2, b2, src2, dst2, zeros2):
    table = proj.reshape(-1, _HID)
    o0, o1 = _sc_gat2(table, a2.reshape(-1), src2, dst2, zeros2)
    s = o0 + o1
    return s[:, :_HID] / (s[:, _HID:_HID + 1] + 1e-16) + b2


def _gnn(xp, p, idx, zeros1, zeros2):
    src4, dst4, src2, dst2 = idx
    p1 = _mm(xp, jnp.concatenate([p['Wl1'], p['Wr1']], axis=1))
    h1 = _gat_layer1(p1, p['a1'], p['b1'], src4, dst4, zeros1)
    p2 = _mm(h1, jnp.concatenate([p['Wl2'], p['Wr2']], axis=1))
    return _gat_layer2(p2, p['a2'], p['b2'], src2, dst2, zeros2)


def kernel(x, edge_type, edge_index, params):
    src = edge_index[0]
    dst = edge_index[1]
    si = jnp.arange(_N, dtype=src.dtype)
    padv = jnp.full((_E1PAD - _EG,), _N, src.dtype)
    srcp = jnp.concatenate([src, si, padv])
    dstp = jnp.concatenate([dst, si, padv])
    src4 = (srcp * 4, srcp * 4 + 1)
    dst4 = (dstp * 4 + 2, dstp * 4 + 3)
    idx = (src4, dst4, srcp * 2, dstp * 2 + 1)
    padc = jnp.full((_ECPAD - _E,), _N, src.dtype)
    srcc = jnp.concatenate([src, padc])
    dstc = jnp.concatenate([dst, padc])

    zeros1 = jnp.zeros((_NPAD, 144), jnp.float32)
    zeros2 = jnp.zeros((_NPAD, 48), jnp.float32)
    zerosc = jnp.zeros((_NPAD, 16), jnp.float32)

    xp = _pad_rows(x)
    ep = _pad_rows(edge_type)
    node_repr = _gnn(xp, params['node'], idx, zeros1, zeros2)
    edge_repr = _gnn(ep, params['edge'], idx, zeros1, zeros2)

    c, pp = params['crf'], params['proxy']
    wf = jnp.concatenate([c['Wu'], pp['W1'], c['Wp']], axis=1)
    wf = jnp.pad(wf, ((0, 0), (0, 128 - wf.shape[1])))
    r = jnp.concatenate([node_repr, edge_repr], axis=0)
    o = _mm(r, wf)
    unary = o[:_N, 0:2] + c['bu']
    hpx = jax.nn.relu(o[:_N, 2:66] + pp['b1'])
    pair = o[_NPAD:_NPAD + _N, 66:68] + c['bp']

    q = jax.nn.softmax(unary, axis=-1)
    t = jnp.zeros((_NPAD, 16), jnp.float32).at[:_N, :2].set(q * pair)
    m0, m1 = _sc_crf(t, srcc, dstc, zerosc)
    crf_out = unary + (m0 + m1)[:_N, :2]

    w2 = jnp.pad(pp['W2'], ((0, 0), (0, 128 - _NCLS)))
    proxy_out = _mm(hpx, w2)[:, :2] + pp['b2']
    return (crf_out, proxy_out)
